# baseline (speedup 1.0000x reference)
import functools

import jax
import jax.numpy as jnp
import numpy as np
from jax import lax
from jax.experimental import pallas as pl
from jax.experimental.pallas import tpu as pltpu
from jax.experimental.pallas import tpu_sc as plsc

N_BINS = 15
ACC = 2048
NW = 32
ROWS_PER_CHUNK = 160
TC_ROWS = 52000
TC_BLOCK = 2000

_BOUNDS = [float(np.float32(t) * np.float32(1.0 / 15.0)) for t in range(16)]


def _sc_body(n_chunks, tc_chunks, sm_hbm, lab_hbm, out_hbm,
             buf0, buf1, lbuf0, lbuf1, conf_v, priv_v,
             s0, s1, sl0, sl1):
    cid = lax.axis_index("c")
    sid = lax.axis_index("s")
    wid = sid * 2 + cid

    zf = jnp.zeros((16,), jnp.float32)

    @plsc.parallel_loop(0, ACC // 16, unroll=8)
    def _(i):
        conf_v[pl.ds(i * 16, 16)] = zf

    @plsc.parallel_loop(0, ACC, unroll=8)
    def _(i):
        priv_v[pl.ds(i * 16, 16)] = zf

    iota = lax.iota(jnp.int32, 16)
    iota16 = iota * 16
    neg_ones = jnp.full((16,), -1.0, jnp.float32)
    tail_mask = iota >= 12
    cvecs = [iota + 16 * vi for vi in range(6)]
    cvec_tail = iota + 84

    bufs = (buf0, buf1)
    lbufs = (lbuf0, lbuf1)
    sems = (s0, s1)
    lsems = (sl0, sl1)

    def start(c, b):
        off = pl.multiple_of(c * ROWS_PER_CHUNK, 8)
        pltpu.async_copy(sm_hbm.at[pl.ds(off, ROWS_PER_CHUNK)], bufs[b], sems[b])
        pltpu.async_copy(lab_hbm.at[pl.ds(off, ROWS_PER_CHUNK)], lbufs[b], lsems[b])

    def wait(b):
        pltpu.make_async_copy(
            sm_hbm.at[pl.ds(0, ROWS_PER_CHUNK)], bufs[b], sems[b]).wait()
        pltpu.make_async_copy(
            lab_hbm.at[pl.ds(0, ROWS_PER_CHUNK)], lbufs[b], lsems[b]).wait()

    def badj_of(x):
        k = (x * 15.0 + 0.5).astype(jnp.int32)
        u = k.astype(jnp.float32) * (1.0 / 15.0)
        return k + jnp.where(u < x, 1, 0)

    def process(c, buf, lbuf):
        @pl.when(c >= tc_chunks)
        def _():
            @plsc.parallel_loop(0, ROWS_PER_CHUNK)
            def _(r):
                for vi in range(6):
                    x = buf[r, pl.ds(vi * 16, 16)]
                    idx = (badj_of(x) << 7) + cvecs[vi]
                    plsc.addupdate_scatter(conf_v, [idx], x)
                x = buf[r, pl.ds(84, 16)]
                idx = (badj_of(x) << 7) + cvec_tail
                plsc.addupdate_scatter(conf_v, [idx], x, mask=tail_mask)

        @plsc.parallel_loop(0, ROWS_PER_CHUNK // 16, unroll=2)
        def _(li):
            lv = lbuf[pl.ds(li * 16, 16)]
            rl = iota + li * 16
            xg = plsc.load_gather(buf, [rl, lv])
            slot = (((badj_of(xg) << 7) + lv) << 4) + iota
            plsc.addupdate_scatter(priv_v, [slot], neg_ones)

    @pl.when(wid < n_chunks)
    def _():
        start(wid, 0)

    @pl.when(wid + NW < n_chunks)
    def _():
        start(wid + NW, 1)

    n_outer = (n_chunks + 2 * NW - 1) // (2 * NW)

    def outer(i, carry):
        for b2 in range(2):
            c = wid + (2 * i + b2) * NW

            @pl.when(c < n_chunks)
            def _():
                wait(b2)
                process(c, bufs[b2], lbufs[b2])
                cn = c + 2 * NW

                @pl.when(cn < n_chunks)
                def _():
                    start(cn, b2)

        return carry

    lax.fori_loop(0, n_outer, outer, 0)

    @plsc.parallel_loop(0, ACC // 16)
    def _(ov):
        s = conf_v[pl.ds(ov * 16, 16)]
        gbase = iota16 + ov * 256
        for l in range(16):
            s = s + plsc.load_gather(priv_v, [gbase + l])
        conf_v[pl.ds(ov * 16, 16)] = s

    pltpu.sync_copy(conf_v, out_hbm.at[wid])


def _thr_body(x_ref, s_ref):
    @pl.when(pl.program_id(0) == 0)
    def _():
        s_ref[...] = jnp.zeros_like(s_ref)

    x = x_ref[...]
    sums = []
    for t in range(16):
        xs = jnp.where(x > _BOUNDS[t], x, 0.0)
        sums.append(jnp.sum(xs, axis=0))
    s_ref[...] += jnp.stack(sums, axis=0)


def _final_body(inv_ncl, p_ref, s_ref, o_ref):
    p = jnp.sum(p_ref[...], axis=0)
    S = s_ref[...]
    Sd = jnp.concatenate([jnp.zeros((1, 100), jnp.float32), S[:15, :]], axis=0)
    tc = Sd - S
    total = p[:, :100] + tc
    r = lax.broadcasted_iota(jnp.int32, (16, 100), 0)
    d = jnp.where(r >= 1, jnp.abs(total), 0.0)
    o_ref[...] = (jnp.sum(d) * inv_ncl).reshape(1, 1)


@jax.jit
def kernel(softmaxes, labels):
    n, num_classes = softmaxes.shape
    assert num_classes == 100 and n % ROWS_PER_CHUNK == 0
    assert TC_ROWS % TC_BLOCK == 0 and TC_ROWS % ROWS_PER_CHUNK == 0
    n_chunks = n // ROWS_PER_CHUNK
    tc_chunks = TC_ROWS // ROWS_PER_CHUNK

    lab = labels.astype(jnp.int32)

    mesh = plsc.VectorSubcoreMesh(core_axis_name="c", subcore_axis_name="s")
    sc = pl.kernel(
        functools.partial(_sc_body, n_chunks, tc_chunks),
        out_type=jax.ShapeDtypeStruct((NW, ACC), jnp.float32),
        mesh=mesh,
        compiler_params=pltpu.CompilerParams(needs_layout_passes=False),
        scratch_types=[
            pltpu.VMEM((ROWS_PER_CHUNK, 100), jnp.float32),
            pltpu.VMEM((ROWS_PER_CHUNK, 100), jnp.float32),
            pltpu.VMEM((ROWS_PER_CHUNK,), jnp.int32),
            pltpu.VMEM((ROWS_PER_CHUNK,), jnp.int32),
            pltpu.VMEM((ACC,), jnp.float32),
            pltpu.VMEM((ACC * 16,), jnp.float32),
            pltpu.SemaphoreType.DMA,
            pltpu.SemaphoreType.DMA,
            pltpu.SemaphoreType.DMA,
            pltpu.SemaphoreType.DMA,
        ],
    )
    part = sc(softmaxes, lab)

    s_tc = pl.pallas_call(
        _thr_body,
        grid=(TC_ROWS // TC_BLOCK,),
        in_specs=[pl.BlockSpec((TC_BLOCK, 100), lambda i: (i, 0))],
        out_specs=pl.BlockSpec((16, 100), lambda i: (0, 0)),
        out_shape=jax.ShapeDtypeStruct((16, 100), jnp.float32),
    )(softmaxes)

    part3 = part.reshape(NW, 16, 128)

    inv_ncl = 1.0 / (float(n) * float(num_classes))
    out = pl.pallas_call(
        functools.partial(_final_body, inv_ncl),
        out_shape=jax.ShapeDtypeStruct((1, 1), jnp.float32),
    )(part3, s_tc)
    return out[0, 0]


# speedup vs baseline: 1.0423x; 1.0423x over previous
import functools

import jax
import jax.numpy as jnp
import numpy as np
from jax import lax
from jax.experimental import pallas as pl
from jax.experimental.pallas import tpu as pltpu
from jax.experimental.pallas import tpu_sc as plsc

N_BINS = 15
ACC = 2048
NW = 32
ROWS_PER_CHUNK = 160
TC_ROWS = 48000
TC_BLOCK = 2000

_BOUNDS = [float(np.float32(t) * np.float32(1.0 / 15.0)) for t in range(16)]


def _sc_body(n_chunks, tc_chunks, sm_hbm, lab_hbm, out_hbm,
             buf0, buf1, lbuf0, lbuf1, conf_v, priv_v,
             s0, s1, sl0, sl1):
    cid = lax.axis_index("c")
    sid = lax.axis_index("s")
    wid = sid * 2 + cid

    zf = jnp.zeros((16,), jnp.float32)

    @plsc.parallel_loop(0, ACC // 16, unroll=8)
    def _(i):
        conf_v[pl.ds(i * 16, 16)] = zf

    @plsc.parallel_loop(0, ACC, unroll=8)
    def _(i):
        priv_v[pl.ds(i * 16, 16)] = zf

    iota = lax.iota(jnp.int32, 16)
    iota16 = iota * 16
    neg_ones = jnp.full((16,), -1.0, jnp.float32)
    tail_mask = iota >= 12
    cvecs = [iota + 16 * vi for vi in range(6)]
    cvec_tail = iota + 84

    bufs = (buf0, buf1)
    lbufs = (lbuf0, lbuf1)
    sems = (s0, s1)
    lsems = (sl0, sl1)

    def start(c, b):
        off = pl.multiple_of(c * ROWS_PER_CHUNK, 8)
        pltpu.async_copy(sm_hbm.at[pl.ds(off, ROWS_PER_CHUNK)], bufs[b], sems[b])
        pltpu.async_copy(lab_hbm.at[pl.ds(off, ROWS_PER_CHUNK)], lbufs[b], lsems[b])

    def wait(b):
        pltpu.make_async_copy(
            sm_hbm.at[pl.ds(0, ROWS_PER_CHUNK)], bufs[b], sems[b]).wait()
        pltpu.make_async_copy(
            lab_hbm.at[pl.ds(0, ROWS_PER_CHUNK)], lbufs[b], lsems[b]).wait()

    def badj_of(x):
        k = (x * 15.0 + 0.5).astype(jnp.int32)
        u = k.astype(jnp.float32) * (1.0 / 15.0)
        return k + jnp.where(u < x, 1, 0)

    def process(c, buf, lbuf):
        @pl.when(c >= tc_chunks)
        def _():
            @plsc.parallel_loop(0, ROWS_PER_CHUNK)
            def _(r):
                for vi in range(6):
                    x = buf[r, pl.ds(vi * 16, 16)]
                    idx = (badj_of(x) << 7) + cvecs[vi]
                    plsc.addupdate_scatter(conf_v, [idx], x)
                x = buf[r, pl.ds(84, 16)]
                idx = (badj_of(x) << 7) + cvec_tail
                plsc.addupdate_scatter(conf_v, [idx], x, mask=tail_mask)

        @plsc.parallel_loop(0, ROWS_PER_CHUNK // 16, unroll=2)
        def _(li):
            lv = lbuf[pl.ds(li * 16, 16)]
            rl = iota + li * 16
            xg = plsc.load_gather(buf, [rl, lv])
            slot = (((badj_of(xg) << 7) + lv) << 4) + iota
            plsc.addupdate_scatter(priv_v, [slot], neg_ones)

    @pl.when(wid < n_chunks)
    def _():
        start(wid, 0)

    @pl.when(wid + NW < n_chunks)
    def _():
        start(wid + NW, 1)

    n_outer = (n_chunks + 2 * NW - 1) // (2 * NW)

    def outer(i, carry):
        for b2 in range(2):
            c = wid + (2 * i + b2) * NW

            @pl.when(c < n_chunks)
            def _():
                wait(b2)
                process(c, bufs[b2], lbufs[b2])
                cn = c + 2 * NW

                @pl.when(cn < n_chunks)
                def _():
                    start(cn, b2)

        return carry

    lax.fori_loop(0, n_outer, outer, 0)

    @plsc.parallel_loop(0, ACC // 16)
    def _(ov):
        s = conf_v[pl.ds(ov * 16, 16)]
        gbase = iota16 + ov * 256
        for l in range(16):
            s = s + plsc.load_gather(priv_v, [gbase + l])
        conf_v[pl.ds(ov * 16, 16)] = s

    pltpu.sync_copy(conf_v, out_hbm.at[wid])


def _thr_body(x_ref, s_ref):
    @pl.when(pl.program_id(0) == 0)
    def _():
        s_ref[...] = jnp.zeros_like(s_ref)

    x = x_ref[...]
    sums = []
    for t in range(16):
        xs = jnp.where(x > _BOUNDS[t], x, 0.0)
        sums.append(jnp.sum(xs, axis=0))
    s_ref[...] += jnp.stack(sums, axis=0)


def _final_body(inv_ncl, p_ref, s_ref, o_ref):
    p = jnp.sum(p_ref[...], axis=0)
    S = s_ref[...]
    Sd = jnp.concatenate([jnp.zeros((1, 100), jnp.float32), S[:15, :]], axis=0)
    tc = Sd - S
    total = p[:, :100] + tc
    r = lax.broadcasted_iota(jnp.int32, (16, 100), 0)
    d = jnp.where(r >= 1, jnp.abs(total), 0.0)
    o_ref[...] = (jnp.sum(d) * inv_ncl).reshape(1, 1)


@jax.jit
def kernel(softmaxes, labels):
    n, num_classes = softmaxes.shape
    assert num_classes == 100 and n % ROWS_PER_CHUNK == 0
    assert TC_ROWS % TC_BLOCK == 0 and TC_ROWS % ROWS_PER_CHUNK == 0
    n_chunks = n // ROWS_PER_CHUNK
    tc_chunks = TC_ROWS // ROWS_PER_CHUNK

    lab = labels.astype(jnp.int32)

    mesh = plsc.VectorSubcoreMesh(core_axis_name="c", subcore_axis_name="s")
    sc = pl.kernel(
        functools.partial(_sc_body, n_chunks, tc_chunks),
        out_type=jax.ShapeDtypeStruct((NW, ACC), jnp.float32),
        mesh=mesh,
        compiler_params=pltpu.CompilerParams(needs_layout_passes=False),
        scratch_types=[
            pltpu.VMEM((ROWS_PER_CHUNK, 100), jnp.float32),
            pltpu.VMEM((ROWS_PER_CHUNK, 100), jnp.float32),
            pltpu.VMEM((ROWS_PER_CHUNK,), jnp.int32),
            pltpu.VMEM((ROWS_PER_CHUNK,), jnp.int32),
            pltpu.VMEM((ACC,), jnp.float32),
            pltpu.VMEM((ACC * 16,), jnp.float32),
            pltpu.SemaphoreType.DMA,
            pltpu.SemaphoreType.DMA,
            pltpu.SemaphoreType.DMA,
            pltpu.SemaphoreType.DMA,
        ],
    )
    part = sc(softmaxes, lab)

    s_tc = pl.pallas_call(
        _thr_body,
        grid=(TC_ROWS // TC_BLOCK,),
        in_specs=[pl.BlockSpec((TC_BLOCK, 100), lambda i: (i, 0))],
        out_specs=pl.BlockSpec((16, 100), lambda i: (0, 0)),
        out_shape=jax.ShapeDtypeStruct((16, 100), jnp.float32),
    )(softmaxes)

    part3 = part.reshape(NW, 16, 128)

    inv_ncl = 1.0 / (float(n) * float(num_classes))
    out = pl.pallas_call(
        functools.partial(_final_body, inv_ncl),
        out_shape=jax.ShapeDtypeStruct((1, 1), jnp.float32),
    )(part3, s_tc)
    return out[0, 0]
